# baseline (speedup 1.0000x reference)
import jax
import jax.numpy as jnp
from jax import lax
from jax.experimental import pallas as pl
from jax.experimental.pallas import tpu as pltpu
from jax.experimental.pallas import tpu_sc as plsc

CAPACITY, GROUPS, HEAD_DIM = 8192, 8, 128
NEW_TOKENS = 32
D = GROUPS * HEAD_DIM

NC, NS = 2, 16
NW = NC * NS
RPW = CAPACITY // NW
CH = 32
NCHUNK = RPW // CH
NB = 2


def _sc_bulk_copy(k_ref, v_ref, ok_ref, ov_ref, buf, sem_in, sem_out):
    wid = lax.axis_index("s") * NC + lax.axis_index("c")
    base = wid * RPW

    ops = ([(k_ref, ok_ref, i) for i in range(NCHUNK)]
           + [(v_ref, ov_ref, i) for i in range(NCHUNK)])
    nop = len(ops)

    def in_copy(j):
        src, _, i = ops[j]
        rows = pl.ds(base + i * CH, CH)
        return pltpu.make_async_copy(src.at[rows], buf.at[j % NB],
                                     sem_in.at[j % NB])

    def out_copy(j):
        _, dst, i = ops[j]
        rows = pl.ds(base + i * CH, CH)
        return pltpu.make_async_copy(buf.at[j % NB], dst.at[rows],
                                     sem_out.at[j % NB])

    in_copy(0).start()
    for j in range(nop):
        in_copy(j).wait()
        out_copy(j).start()
        if j + 1 < nop:
            if j + 1 >= NB:
                out_copy(j + 1 - NB).wait()
            in_copy(j + 1).start()
    for j in range(nop - NB, nop):
        out_copy(j).wait()


def _tc_slab_update(cur_ref, ok_in, ov_in, ak_ref, av_ref, ok_ref, ov_ref,
                    akbuf, avbuf, sems):
    del ok_in, ov_in
    cur = jnp.clip(cur_ref[0], 0, CAPACITY - NEW_TOKENS)
    cur = pl.multiple_of(cur, 8)

    ins = [pltpu.make_async_copy(ak_ref, akbuf, sems.at[0]),
           pltpu.make_async_copy(av_ref, avbuf, sems.at[1])]
    for c in ins:
        c.start()
    for c in ins:
        c.wait()
    sl = pl.ds(cur, NEW_TOKENS)
    outs = [pltpu.make_async_copy(akbuf, ok_ref.at[sl], sems.at[2]),
            pltpu.make_async_copy(avbuf, ov_ref.at[sl], sems.at[3])]
    for c in outs:
        c.start()
    for c in outs:
        c.wait()


def kernel(keys, values, added_keys, added_values, current_length):
    num_added = added_keys.shape[0]
    k2 = keys.reshape(CAPACITY, D)
    v2 = values.reshape(CAPACITY, D)
    ak2 = added_keys.reshape(NEW_TOKENS, D)
    av2 = added_values.reshape(NEW_TOKENS, D)
    cur1 = jnp.reshape(current_length, (1,)).astype(jnp.int32)

    sc = pl.kernel(
        _sc_bulk_copy,
        out_type=(
            jax.ShapeDtypeStruct((CAPACITY, D), jnp.float32),
            jax.ShapeDtypeStruct((CAPACITY, D), jnp.float32),
        ),
        mesh=plsc.VectorSubcoreMesh(core_axis_name="c", subcore_axis_name="s"),
        scratch_types=[
            pltpu.VMEM((NB, CH, D), jnp.float32),
            pltpu.SemaphoreType.DMA((NB,)),
            pltpu.SemaphoreType.DMA((NB,)),
        ],
    )
    bk, bv = sc(k2, v2)

    hbm = pl.BlockSpec(memory_space=pltpu.MemorySpace.HBM)
    ok, ov = pl.pallas_call(
        _tc_slab_update,
        in_specs=[pl.BlockSpec(memory_space=pltpu.SMEM), hbm, hbm, hbm, hbm],
        out_specs=(hbm, hbm),
        out_shape=(
            jax.ShapeDtypeStruct((CAPACITY, D), jnp.float32),
            jax.ShapeDtypeStruct((CAPACITY, D), jnp.float32),
        ),
        input_output_aliases={1: 0, 2: 1},
        scratch_shapes=[
            pltpu.VMEM((NEW_TOKENS, D), jnp.float32),
            pltpu.VMEM((NEW_TOKENS, D), jnp.float32),
            pltpu.SemaphoreType.DMA((4,)),
        ],
    )(cur1, bk, bv, ak2, av2)
    return (ok.reshape(CAPACITY, GROUPS, HEAD_DIM),
            ov.reshape(CAPACITY, GROUPS, HEAD_DIM),
            current_length + num_added)
